# Initial kernel scaffold; baseline (speedup 1.0000x reference)
import jax
import jax.numpy as jnp
from jax.experimental import pallas as pl


def kernel(xyz, point_feature, patch_center, W1, b1, W2, b2):
    raise NotImplementedError("write your pallas kernel here")


# speedup vs baseline: 7.0054x; 7.0054x over previous
import dataclasses
import functools

import jax
import jax.numpy as jnp
import numpy as np
from jax import lax
from jax.experimental import pallas as pl
from jax.experimental.pallas import tpu as pltpu
from jax.experimental.pallas import tpu_sc as plsc

RADIUS = 0.2
NSAMPLE = 32
PE_DIM = 24
STEM = 128
PATCH = 256

B, N, S = 4, 8192, 512
GW = 256
NC, NS_SC, L = 2, 16, 16
NW = NC * NS_SC
CPW = (B * S) // NW
SBLK = S // (NW // B)
NVEC = N // L
GCHUNK = 256
ROWS_PW = (B * S * NSAMPLE) // NW


def _bf16r(v):
    u = plsc.bitcast(v, jnp.int32)
    r = (u + 0x7FFF + jnp.bitwise_and(jnp.right_shift(u, 16), 1)) & jnp.int32(-65536)
    return plsc.bitcast(r, jnp.float32)


def _sc_select(xyzT, centT):
    mesh = plsc.VectorSubcoreMesh(core_axis_name="c", subcore_axis_name="s")
    r2 = np.float32(RADIUS * RADIUS)
    cp = pltpu.CompilerParams()
    if "needs_layout_passes" in pltpu.CompilerParams.__dataclass_fields__:
        cp = dataclasses.replace(cp, needs_layout_passes=False)

    @functools.partial(
        pl.kernel,
        compiler_params=cp,
        out_type=(
            jax.ShapeDtypeStruct((B, S, NSAMPLE), jnp.int32),
            jax.ShapeDtypeStruct((B * S, NSAMPLE), jnp.int32),
        ),
        mesh=mesh,
        scratch_types=[
            pltpu.VMEM((1, N), jnp.float32),
            pltpu.VMEM((1, N), jnp.float32),
            pltpu.VMEM((1, N), jnp.float32),
            pltpu.VMEM((1, N), jnp.float32),
            pltpu.VMEM((1, N), jnp.float32),
            pltpu.VMEM((1, N), jnp.float32),
            pltpu.VMEM((1, N), jnp.float32),
            pltpu.VMEM((1, S), jnp.float32),
            pltpu.VMEM((1, S), jnp.float32),
            pltpu.VMEM((1, S), jnp.float32),
            pltpu.VMEM((128,), jnp.int32),
            pltpu.VMEM((SBLK, NSAMPLE), jnp.int32),
            pltpu.VMEM((SBLK, NSAMPLE), jnp.int32),
        ],
    )
    def sel_kernel(xyz_hbm, cent_hbm, nbr_hbm, flat_hbm,
                   xv, yv, zv, nv, xrv, yrv, zrv, cxv, cyv, czv, cand, outv,
                   flatv):
        wid = lax.axis_index("s") * NC + lax.axis_index("c")
        b = wid // (NW // B)
        s0 = (wid % (NW // B)) * SBLK
        pltpu.sync_copy(xyz_hbm.at[pl.ds(b * 3 + 0, 1)], xv)
        pltpu.sync_copy(xyz_hbm.at[pl.ds(b * 3 + 1, 1)], yv)
        pltpu.sync_copy(xyz_hbm.at[pl.ds(b * 3 + 2, 1)], zv)
        pltpu.sync_copy(cent_hbm.at[pl.ds(b * 3 + 0, 1)], cxv)
        pltpu.sync_copy(cent_hbm.at[pl.ds(b * 3 + 1, 1)], cyv)
        pltpu.sync_copy(cent_hbm.at[pl.ds(b * 3 + 2, 1)], czv)

        @pl.loop(0, N, step=L)
        def _(i):
            x = xv[0, pl.ds(i, L)]
            y = yv[0, pl.ds(i, L)]
            z = zv[0, pl.ds(i, L)]
            nv[0, pl.ds(i, L)] = (x * x + y * y) + z * z
            xrv[0, pl.ds(i, L)] = _bf16r(x)
            yrv[0, pl.ds(i, L)] = _bf16r(y)
            zrv[0, pl.ds(i, L)] = _bf16r(z)

        @pl.loop(0, SBLK)
        def _center(j):
            jv = jnp.full((L,), j + s0, jnp.int32)
            zv16 = jnp.zeros((L,), jnp.int32)
            cx = plsc.load_gather(cxv, [zv16, jv])
            cy = plsc.load_gather(cyv, [zv16, jv])
            cz = plsc.load_gather(czv, [zv16, jv])
            cn = (cx * cx + cy * cy) + cz * cz
            cxr = _bf16r(cx)
            cyr = _bf16r(cy)
            czr = _bf16r(cz)

            def cond(c):
                i, cur = c
                return jnp.logical_and(cur < NSAMPLE, i < NVEC)

            def body(c):
                i, cur = c
                nb = i * L
                x = xrv[0, pl.ds(nb, L)]
                y = yrv[0, pl.ds(nb, L)]
                z = zrv[0, pl.ds(nb, L)]
                xn = nv[0, pl.ds(nb, L)]
                dot = (cxr * x + cyr * y) + czr * z
                d2 = (np.float32(-2.0) * dot + cn) + xn
                m = d2 <= r2
                mi = m.astype(jnp.int32)
                rank = jnp.cumsum(mi)
                pos = rank + (cur - 1)
                gidx = lax.iota(jnp.int32, L) + nb
                plsc.store_scatter(cand, [pos], gidx, mask=m)
                return i + 1, cur + jnp.sum(mi)

            _, found = lax.while_loop(cond, body, (0, 0))
            foundv = jnp.full((L,), jnp.minimum(found, NSAMPLE), jnp.int32)
            first = plsc.load_gather(cand, [jnp.zeros((L,), jnp.int32)])
            firstv = jnp.where(foundv > 0, first, jnp.full((L,), N, jnp.int32))
            bo = jnp.full((L,), b * N, jnp.int32)
            for v in range(NSAMPLE // L):
                kio = lax.iota(jnp.int32, L) + v * L
                cur = cand[pl.ds(v * L, L)]
                sel = jnp.where(kio < foundv, cur, firstv)
                outv[j, pl.ds(v * L, L)] = sel
                flatv[j, pl.ds(v * L, L)] = jnp.minimum(sel, N - 1) + bo

        pltpu.sync_copy(outv, nbr_hbm.at[b, pl.ds(s0, SBLK)])
        pltpu.sync_copy(flatv, flat_hbm.at[pl.ds(b * S + s0, SBLK)])

    return sel_kernel(xyzT, centT)


def _sc_gather(table, flat_idx):
    mesh = plsc.VectorSubcoreMesh(core_axis_name="c", subcore_axis_name="s")

    @functools.partial(
        pl.kernel,
        out_type=jax.ShapeDtypeStruct((B * S * NSAMPLE, GW), jnp.float32),
        mesh=mesh,
        scratch_types=[
            pltpu.VMEM((ROWS_PW,), jnp.int32),
            pltpu.VMEM((GCHUNK, GW), jnp.float32),
            pltpu.SemaphoreType.DMA,
        ],
    )
    def gather_kernel_seq(table_hbm, idx_hbm, out_hbm, idxv, rowsv, sem):
        wid = lax.axis_index("s") * NC + lax.axis_index("c")
        base = wid * ROWS_PW
        pltpu.sync_copy(idx_hbm.at[pl.ds(base, ROWS_PW)], idxv)

        @pl.loop(0, ROWS_PW // GCHUNK)
        def _(c):
            pltpu.async_copy(
                table_hbm.at[idxv.at[pl.ds(c * GCHUNK, GCHUNK)]],
                rowsv, sem).wait()
            pltpu.sync_copy(rowsv, out_hbm.at[pl.ds(base + c * GCHUNK, GCHUNK)])

    return gather_kernel_seq(table, flat_idx)


def _mlp_body(g_ref, c_ref, w1f_ref, w1rp_ref, b1_ref, w2_ref, b2_ref, o_ref):
    g = g_ref[...]
    feat = g[:, :STEM]
    rel = g[:, STEM:STEM + 3] - c_ref[...]
    nf = PE_DIM // 6
    fr = jnp.exp2(lax.broadcasted_iota(jnp.int32, (1, nf), 1).astype(jnp.float32)
                  ) * np.float32(np.pi)
    parts = []
    for c in range(3):
        ang = rel[:, c:c + 1] * fr
        parts.append(jnp.sin(ang))
        parts.append(jnp.cos(ang))
    parts.append(rel)
    rp = jnp.concatenate(parts, axis=1)
    h = (
        jax.lax.dot_general(feat, w1f_ref[...], (((1,), (0,)), ((), ())),
                            precision=lax.Precision.HIGHEST,
                            preferred_element_type=jnp.float32)
        + jax.lax.dot_general(rp, w1rp_ref[...], (((1,), (0,)), ((), ())),
                              precision=lax.Precision.HIGHEST,
                              preferred_element_type=jnp.float32)
        + b1_ref[...]
    )
    h = jnp.maximum(h, 0.0)
    o = jax.lax.dot_general(h, w2_ref[...], (((1,), (0,)), ((), ())),
                            precision=lax.Precision.HIGHEST,
                            preferred_element_type=jnp.float32) + b2_ref[...]
    o3 = o.reshape(o.shape[0] // NSAMPLE, NSAMPLE, PATCH)
    o_ref[...] = jnp.max(o3, axis=1)


RBLK = 2048


def _tc_mlp(gathered, crep, W1f, W1rp, b1, W2, b2):
    nrow = B * S * NSAMPLE
    grid = (nrow // RBLK,)
    return pl.pallas_call(
        _mlp_body,
        grid=grid,
        in_specs=[
            pl.BlockSpec((RBLK, GW), lambda i: (i, 0)),
            pl.BlockSpec((RBLK, 3), lambda i: (i, 0)),
            pl.BlockSpec((STEM, PATCH), lambda i: (0, 0)),
            pl.BlockSpec((PE_DIM + 3, PATCH), lambda i: (0, 0)),
            pl.BlockSpec((1, PATCH), lambda i: (0, 0)),
            pl.BlockSpec((PATCH, PATCH), lambda i: (0, 0)),
            pl.BlockSpec((1, PATCH), lambda i: (0, 0)),
        ],
        out_specs=pl.BlockSpec((RBLK // NSAMPLE, PATCH), lambda i: (i, 0)),
        out_shape=jax.ShapeDtypeStruct((B * S, PATCH), jnp.float32),
    )(gathered, crep, W1f, W1rp, b1, W2, b2)


def kernel(xyz, point_feature, patch_center, W1, b1, W2, b2):
    xyzT = jnp.transpose(xyz, (0, 2, 1)).reshape(B * 3, N)
    centT = jnp.transpose(patch_center, (0, 2, 1)).reshape(B * 3, S)
    neighbor_idx, flat_idx = _sc_select(xyzT, centT)

    pad = jnp.zeros((B, N, GW - STEM - 3), jnp.float32)
    table = jnp.concatenate([point_feature, xyz, pad], axis=-1).reshape(B * N, GW)
    rows = _sc_gather(table, flat_idx.reshape(-1))

    crep = jnp.repeat(patch_center.reshape(B * S, 3), NSAMPLE, axis=0)
    W1f = W1[:STEM]
    W1rp = jnp.concatenate([W1[STEM + 3:], W1[STEM:STEM + 3]], axis=0)
    pf = _tc_mlp(rows, crep, W1f, W1rp, b1.reshape(1, PATCH), W2,
                 b2.reshape(1, PATCH))
    return pf.reshape(B, S, PATCH), neighbor_idx


# speedup vs baseline: 9.7962x; 1.3984x over previous
import dataclasses
import functools

import jax
import jax.numpy as jnp
import numpy as np
from jax import lax
from jax.experimental import pallas as pl
from jax.experimental.pallas import tpu as pltpu
from jax.experimental.pallas import tpu_sc as plsc

RADIUS = 0.2
NSAMPLE = 32
PE_DIM = 24
STEM = 128
PATCH = 256

B, N, S = 4, 8192, 512
GW = 128
NC, NS_SC, L = 2, 16, 16
NW = NC * NS_SC
CPW = (B * S) // NW
SBLK = S // (NW // B)
NVEC = N // L
GCHUNK = 256
ROWS_PW = (B * S * NSAMPLE) // NW


def _bf16r(v):
    u = plsc.bitcast(v, jnp.int32)
    r = (u + 0x7FFF + jnp.bitwise_and(jnp.right_shift(u, 16), 1)) & jnp.int32(-65536)
    return plsc.bitcast(r, jnp.float32)


def _sc_select(xyzT, centT):
    mesh = plsc.VectorSubcoreMesh(core_axis_name="c", subcore_axis_name="s")
    r2 = np.float32(RADIUS * RADIUS)
    cp = pltpu.CompilerParams()
    if "needs_layout_passes" in pltpu.CompilerParams.__dataclass_fields__:
        cp = dataclasses.replace(cp, needs_layout_passes=False)

    @functools.partial(
        pl.kernel,
        compiler_params=cp,
        out_type=(
            jax.ShapeDtypeStruct((B, S, NSAMPLE), jnp.int32),
            jax.ShapeDtypeStruct((B * S, NSAMPLE), jnp.int32),
            jax.ShapeDtypeStruct((B * S, 3 * NSAMPLE), jnp.float32),
        ),
        mesh=mesh,
        scratch_types=[
            pltpu.VMEM((1, N), jnp.float32),
            pltpu.VMEM((1, N), jnp.float32),
            pltpu.VMEM((1, N), jnp.float32),
            pltpu.VMEM((1, N), jnp.float32),
            pltpu.VMEM((1, N), jnp.float32),
            pltpu.VMEM((1, N), jnp.float32),
            pltpu.VMEM((1, N), jnp.float32),
            pltpu.VMEM((1, S), jnp.float32),
            pltpu.VMEM((1, S), jnp.float32),
            pltpu.VMEM((1, S), jnp.float32),
            pltpu.VMEM((128,), jnp.int32),
            pltpu.VMEM((SBLK, NSAMPLE), jnp.int32),
            pltpu.VMEM((SBLK, NSAMPLE), jnp.int32),
            pltpu.VMEM((SBLK, 3 * NSAMPLE), jnp.float32),
        ],
    )
    def sel_kernel(xyz_hbm, cent_hbm, nbr_hbm, flat_hbm, rel_hbm,
                   xv, yv, zv, nv, xrv, yrv, zrv, cxv, cyv, czv, cand, outv,
                   flatv, relv):
        wid = lax.axis_index("s") * NC + lax.axis_index("c")
        b = wid // (NW // B)
        s0 = (wid % (NW // B)) * SBLK
        pltpu.sync_copy(xyz_hbm.at[pl.ds(b * 3 + 0, 1)], xv)
        pltpu.sync_copy(xyz_hbm.at[pl.ds(b * 3 + 1, 1)], yv)
        pltpu.sync_copy(xyz_hbm.at[pl.ds(b * 3 + 2, 1)], zv)
        pltpu.sync_copy(cent_hbm.at[pl.ds(b * 3 + 0, 1)], cxv)
        pltpu.sync_copy(cent_hbm.at[pl.ds(b * 3 + 1, 1)], cyv)
        pltpu.sync_copy(cent_hbm.at[pl.ds(b * 3 + 2, 1)], czv)

        @pl.loop(0, N, step=L)
        def _(i):
            x = xv[0, pl.ds(i, L)]
            y = yv[0, pl.ds(i, L)]
            z = zv[0, pl.ds(i, L)]
            nv[0, pl.ds(i, L)] = (x * x + y * y) + z * z
            xrv[0, pl.ds(i, L)] = _bf16r(x)
            yrv[0, pl.ds(i, L)] = _bf16r(y)
            zrv[0, pl.ds(i, L)] = _bf16r(z)

        @pl.loop(0, SBLK)
        def _center(j):
            jv = jnp.full((L,), j + s0, jnp.int32)
            zv16 = jnp.zeros((L,), jnp.int32)
            cx = plsc.load_gather(cxv, [zv16, jv])
            cy = plsc.load_gather(cyv, [zv16, jv])
            cz = plsc.load_gather(czv, [zv16, jv])
            cn = (cx * cx + cy * cy) + cz * cz
            cxr = _bf16r(cx)
            cyr = _bf16r(cy)
            czr = _bf16r(cz)

            def cond(c):
                i, cur = c
                return jnp.logical_and(cur < NSAMPLE, i < NVEC)

            def body(c):
                i, cur = c
                nb = i * L
                x = xrv[0, pl.ds(nb, L)]
                y = yrv[0, pl.ds(nb, L)]
                z = zrv[0, pl.ds(nb, L)]
                xn = nv[0, pl.ds(nb, L)]
                dot = (cxr * x + cyr * y) + czr * z
                d2 = (np.float32(-2.0) * dot + cn) + xn
                m = d2 <= r2
                mi = m.astype(jnp.int32)
                rank = jnp.cumsum(mi)
                pos = rank + (cur - 1)
                gidx = lax.iota(jnp.int32, L) + nb
                plsc.store_scatter(cand, [pos], gidx, mask=m)
                return i + 1, cur + jnp.sum(mi)

            _, found = lax.while_loop(cond, body, (0, 0))
            foundv = jnp.full((L,), jnp.minimum(found, NSAMPLE), jnp.int32)
            first = plsc.load_gather(cand, [jnp.zeros((L,), jnp.int32)])
            firstv = jnp.where(foundv > 0, first, jnp.full((L,), N, jnp.int32))
            bo = jnp.full((L,), b * N, jnp.int32)
            for v in range(NSAMPLE // L):
                kio = lax.iota(jnp.int32, L) + v * L
                cur = cand[pl.ds(v * L, L)]
                sel = jnp.where(kio < foundv, cur, firstv)
                clp = jnp.minimum(sel, N - 1)
                outv[j, pl.ds(v * L, L)] = sel
                flatv[j, pl.ds(v * L, L)] = clp + bo
                gx = plsc.load_gather(xv, [zv16, clp])
                gy = plsc.load_gather(yv, [zv16, clp])
                gz = plsc.load_gather(zv, [zv16, clp])
                relv[j, pl.ds(0 * NSAMPLE + v * L, L)] = gx - cx
                relv[j, pl.ds(1 * NSAMPLE + v * L, L)] = gy - cy
                relv[j, pl.ds(2 * NSAMPLE + v * L, L)] = gz - cz

        pltpu.sync_copy(outv, nbr_hbm.at[b, pl.ds(s0, SBLK)])
        pltpu.sync_copy(flatv, flat_hbm.at[pl.ds(b * S + s0, SBLK)])
        pltpu.sync_copy(relv, rel_hbm.at[pl.ds(b * S + s0, SBLK)])

    return sel_kernel(xyzT, centT)


def _sc_gather(table, flat_idx):
    mesh = plsc.VectorSubcoreMesh(core_axis_name="c", subcore_axis_name="s")

    @functools.partial(
        pl.kernel,
        out_type=jax.ShapeDtypeStruct((B * S * NSAMPLE, GW), jnp.float32),
        mesh=mesh,
        scratch_types=[
            pltpu.VMEM((ROWS_PW,), jnp.int32),
            pltpu.VMEM((GCHUNK, GW), jnp.float32),
            pltpu.SemaphoreType.DMA,
        ],
    )
    def gather_kernel_seq(table_hbm, idx_hbm, out_hbm, idxv, rowsv, sem):
        wid = lax.axis_index("s") * NC + lax.axis_index("c")
        base = wid * ROWS_PW
        pltpu.sync_copy(idx_hbm.at[pl.ds(base, ROWS_PW)], idxv)

        @pl.loop(0, ROWS_PW // GCHUNK)
        def _(c):
            pltpu.async_copy(
                table_hbm.at[idxv.at[pl.ds(c * GCHUNK, GCHUNK)]],
                rowsv, sem).wait()
            pltpu.sync_copy(rowsv, out_hbm.at[pl.ds(base + c * GCHUNK, GCHUNK)])

    return gather_kernel_seq(table, flat_idx)


def _mlp_body(g_ref, r_ref, w1f_ref, w1rp_ref, b1_ref, w2_ref, b2_ref, o_ref):
    feat = g_ref[...]
    rel = r_ref[...]
    nf = PE_DIM // 6
    fr = jnp.exp2(lax.broadcasted_iota(jnp.int32, (1, nf), 1).astype(jnp.float32)
                  ) * np.float32(np.pi)
    parts = []
    for c in range(3):
        ang = rel[:, c:c + 1] * fr
        parts.append(jnp.sin(ang))
        parts.append(jnp.cos(ang))
    parts.append(rel)
    rp = jnp.concatenate(parts, axis=1)
    h = (
        jax.lax.dot_general(feat, w1f_ref[...], (((1,), (0,)), ((), ())),
                            preferred_element_type=jnp.float32)
        + jax.lax.dot_general(rp, w1rp_ref[...], (((1,), (0,)), ((), ())),
                              preferred_element_type=jnp.float32)
        + b1_ref[...]
    )
    h = jnp.maximum(h, 0.0)
    o = jax.lax.dot_general(h, w2_ref[...], (((1,), (0,)), ((), ())),
                            preferred_element_type=jnp.float32) + b2_ref[...]
    o3 = o.reshape(o.shape[0] // NSAMPLE, NSAMPLE, PATCH)
    o_ref[...] = jnp.max(o3, axis=1)


RBLK = 2048


def _tc_mlp(gathered, crep, W1f, W1rp, b1, W2, b2):
    nrow = B * S * NSAMPLE
    grid = (nrow // RBLK,)
    return pl.pallas_call(
        _mlp_body,
        grid=grid,
        in_specs=[
            pl.BlockSpec((RBLK, STEM), lambda i: (i, 0)),
            pl.BlockSpec((RBLK, 3), lambda i: (i, 0)),
            pl.BlockSpec((STEM, PATCH), lambda i: (0, 0)),
            pl.BlockSpec((PE_DIM + 3, PATCH), lambda i: (0, 0)),
            pl.BlockSpec((1, PATCH), lambda i: (0, 0)),
            pl.BlockSpec((PATCH, PATCH), lambda i: (0, 0)),
            pl.BlockSpec((1, PATCH), lambda i: (0, 0)),
        ],
        out_specs=pl.BlockSpec((RBLK // NSAMPLE, PATCH), lambda i: (i, 0)),
        out_shape=jax.ShapeDtypeStruct((B * S, PATCH), jnp.float32),
    )(gathered, crep, W1f, W1rp, b1, W2, b2)


def kernel(xyz, point_feature, patch_center, W1, b1, W2, b2):
    xyzT = jnp.transpose(xyz, (0, 2, 1)).reshape(B * 3, N)
    centT = jnp.transpose(patch_center, (0, 2, 1)).reshape(B * 3, S)
    neighbor_idx, flat_idx, rel = _sc_select(xyzT, centT)

    table = point_feature.reshape(B * N, STEM)
    rows = _sc_gather(table, flat_idx.reshape(-1))

    rel_flat = jnp.transpose(rel.reshape(B * S, 3, NSAMPLE),
                             (0, 2, 1)).reshape(B * S * NSAMPLE, 3)
    W1f = W1[:STEM]
    W1rp = jnp.concatenate([W1[STEM + 3:], W1[STEM:STEM + 3]], axis=0)
    pf = _tc_mlp(rows, rel_flat, W1f, W1rp, b1.reshape(1, PATCH), W2,
                 b2.reshape(1, PATCH))
    return pf.reshape(B, S, PATCH), neighbor_idx


# speedup vs baseline: 16.0165x; 1.6350x over previous
import dataclasses
import functools

import jax
import jax.numpy as jnp
import numpy as np
from jax import lax
from jax.experimental import pallas as pl
from jax.experimental.pallas import tpu as pltpu
from jax.experimental.pallas import tpu_sc as plsc

RADIUS = 0.2
NSAMPLE = 32
PE_DIM = 24
STEM = 128
PATCH = 256

B, N, S = 4, 8192, 512
GW = 128
NC, NS_SC, L = 2, 16, 16
NW = NC * NS_SC
CPW = (B * S) // NW
SBLK = S // (NW // B)
NVEC = N // L
GCHUNK = 256
ROWS_PW = (B * S * NSAMPLE) // NW


def _bf16r(v):
    u = plsc.bitcast(v, jnp.int32)
    r = (u + 0x7FFF + jnp.bitwise_and(jnp.right_shift(u, 16), 1)) & jnp.int32(-65536)
    return plsc.bitcast(r, jnp.float32)


def _sc_select(xyzT, centT):
    mesh = plsc.VectorSubcoreMesh(core_axis_name="c", subcore_axis_name="s")
    r2 = np.float32(RADIUS * RADIUS)
    cp = pltpu.CompilerParams()
    if "needs_layout_passes" in pltpu.CompilerParams.__dataclass_fields__:
        cp = dataclasses.replace(cp, needs_layout_passes=False)

    @functools.partial(
        pl.kernel,
        compiler_params=cp,
        out_type=(
            jax.ShapeDtypeStruct((B, S, NSAMPLE), jnp.int32),
            jax.ShapeDtypeStruct((B * S, NSAMPLE), jnp.int32),
            jax.ShapeDtypeStruct((B * S, 3 * NSAMPLE), jnp.float32),
        ),
        mesh=mesh,
        scratch_types=[
            pltpu.VMEM((1, N), jnp.float32),
            pltpu.VMEM((1, N), jnp.float32),
            pltpu.VMEM((1, N), jnp.float32),
            pltpu.VMEM((1, N), jnp.float32),
            pltpu.VMEM((1, N), jnp.float32),
            pltpu.VMEM((1, N), jnp.float32),
            pltpu.VMEM((1, N), jnp.float32),
            pltpu.VMEM((1, S), jnp.float32),
            pltpu.VMEM((1, S), jnp.float32),
            pltpu.VMEM((1, S), jnp.float32),
            pltpu.VMEM((128,), jnp.int32),
            pltpu.VMEM((SBLK, NSAMPLE), jnp.int32),
            pltpu.VMEM((SBLK, NSAMPLE), jnp.int32),
            pltpu.VMEM((SBLK, 3 * NSAMPLE), jnp.float32),
        ],
    )
    def sel_kernel(xyz_hbm, cent_hbm, nbr_hbm, flat_hbm, rel_hbm,
                   xv, yv, zv, nv, xrv, yrv, zrv, cxv, cyv, czv, cand, outv,
                   flatv, relv):
        wid = lax.axis_index("s") * NC + lax.axis_index("c")
        b = wid // (NW // B)
        s0 = (wid % (NW // B)) * SBLK
        pltpu.sync_copy(xyz_hbm.at[pl.ds(b * 3 + 0, 1)], xv)
        pltpu.sync_copy(xyz_hbm.at[pl.ds(b * 3 + 1, 1)], yv)
        pltpu.sync_copy(xyz_hbm.at[pl.ds(b * 3 + 2, 1)], zv)
        pltpu.sync_copy(cent_hbm.at[pl.ds(b * 3 + 0, 1)], cxv)
        pltpu.sync_copy(cent_hbm.at[pl.ds(b * 3 + 1, 1)], cyv)
        pltpu.sync_copy(cent_hbm.at[pl.ds(b * 3 + 2, 1)], czv)

        @pl.loop(0, N, step=L)
        def _(i):
            x = xv[0, pl.ds(i, L)]
            y = yv[0, pl.ds(i, L)]
            z = zv[0, pl.ds(i, L)]
            nv[0, pl.ds(i, L)] = (x * x + y * y) + z * z
            xrv[0, pl.ds(i, L)] = _bf16r(x)
            yrv[0, pl.ds(i, L)] = _bf16r(y)
            zrv[0, pl.ds(i, L)] = _bf16r(z)

        @pl.loop(0, SBLK)
        def _center(j):
            jv = jnp.full((L,), j + s0, jnp.int32)
            zv16 = jnp.zeros((L,), jnp.int32)
            cx = plsc.load_gather(cxv, [zv16, jv])
            cy = plsc.load_gather(cyv, [zv16, jv])
            cz = plsc.load_gather(czv, [zv16, jv])
            cn = (cx * cx + cy * cy) + cz * cz
            cxr = _bf16r(cx)
            cyr = _bf16r(cy)
            czr = _bf16r(cz)

            def cond(c):
                i, cur = c
                return jnp.logical_and(cur < NSAMPLE, i < NVEC)

            def body(c):
                i, cur = c
                nb = i * L
                x = xrv[0, pl.ds(nb, L)]
                y = yrv[0, pl.ds(nb, L)]
                z = zrv[0, pl.ds(nb, L)]
                xn = nv[0, pl.ds(nb, L)]
                dot = (cxr * x + cyr * y) + czr * z
                d2 = (np.float32(-2.0) * dot + cn) + xn
                m = d2 <= r2
                mi = m.astype(jnp.int32)
                rank = jnp.cumsum(mi)
                pos = rank + (cur - 1)
                gidx = lax.iota(jnp.int32, L) + nb
                plsc.store_scatter(cand, [pos], gidx, mask=m)
                return i + 1, cur + jnp.sum(mi)

            _, found = lax.while_loop(cond, body, (0, 0))
            foundv = jnp.full((L,), jnp.minimum(found, NSAMPLE), jnp.int32)
            first = plsc.load_gather(cand, [jnp.zeros((L,), jnp.int32)])
            firstv = jnp.where(foundv > 0, first, jnp.full((L,), N, jnp.int32))
            bo = jnp.full((L,), b * N, jnp.int32)
            for v in range(NSAMPLE // L):
                kio = lax.iota(jnp.int32, L) + v * L
                cur = cand[pl.ds(v * L, L)]
                sel = jnp.where(kio < foundv, cur, firstv)
                clp = jnp.minimum(sel, N - 1)
                outv[j, pl.ds(v * L, L)] = sel
                flatv[j, pl.ds(v * L, L)] = clp + bo
                gx = plsc.load_gather(xv, [zv16, clp])
                gy = plsc.load_gather(yv, [zv16, clp])
                gz = plsc.load_gather(zv, [zv16, clp])
                relv[j, pl.ds(0 * NSAMPLE + v * L, L)] = gx - cx
                relv[j, pl.ds(1 * NSAMPLE + v * L, L)] = gy - cy
                relv[j, pl.ds(2 * NSAMPLE + v * L, L)] = gz - cz

        pltpu.sync_copy(outv, nbr_hbm.at[b, pl.ds(s0, SBLK)])
        pltpu.sync_copy(flatv, flat_hbm.at[pl.ds(b * S + s0, SBLK)])
        pltpu.sync_copy(relv, rel_hbm.at[pl.ds(b * S + s0, SBLK)])

    return sel_kernel(xyzT, centT)


def _sc_gather(table, flat_idx):
    mesh = plsc.VectorSubcoreMesh(core_axis_name="c", subcore_axis_name="s")

    @functools.partial(
        pl.kernel,
        out_type=jax.ShapeDtypeStruct((B * S * NSAMPLE, GW), jnp.float32),
        mesh=mesh,
        scratch_types=[
            pltpu.VMEM((ROWS_PW,), jnp.int32),
            pltpu.VMEM((GCHUNK, GW), jnp.float32),
            pltpu.SemaphoreType.DMA,
        ],
    )
    def gather_kernel_seq(table_hbm, idx_hbm, out_hbm, idxv, rowsv, sem):
        wid = lax.axis_index("s") * NC + lax.axis_index("c")
        base = wid * ROWS_PW
        pltpu.sync_copy(idx_hbm.at[pl.ds(base, ROWS_PW)], idxv)

        @pl.loop(0, ROWS_PW // GCHUNK)
        def _(c):
            pltpu.async_copy(
                table_hbm.at[idxv.at[pl.ds(c * GCHUNK, GCHUNK)]],
                rowsv, sem).wait()
            pltpu.sync_copy(rowsv, out_hbm.at[pl.ds(base + c * GCHUNK, GCHUNK)])

    return gather_kernel_seq(table, flat_idx)


def _mlp_body(g_ref, r_ref, w1f_ref, w1rp_ref, b1_ref, w2_ref, b2_ref, o_ref):
    feat = g_ref[...]
    rel = r_ref[...]
    nf = PE_DIM // 6
    f4 = jnp.exp2(lax.broadcasted_iota(jnp.int32, (1, nf), 1).astype(jnp.float32))
    t = jnp.concatenate([rel[:, c:c + 1] * f4 for c in range(3)], axis=1)
    n = jnp.floor(t + np.float32(0.5))
    r = t - n
    r2 = r * r
    ps = ((((np.float32(0.07744687795639038) * r2
             + np.float32(-0.5981614589691162)) * r2
            + np.float32(2.550050973892212)) * r2
           + np.float32(-5.167707920074463)) * r2
          + np.float32(3.141592502593994)) * r
    pc = (((((np.float32(-0.024456139653921127) * r2
              + np.float32(0.2349717617034912)) * r2
             + np.float32(-1.335218906402588)) * r2
            + np.float32(4.058709621429443)) * r2
           + np.float32(-4.934802055358887)) * r2
          + np.float32(1.0))
    odd = jnp.bitwise_and(n.astype(jnp.int32), 1)
    sgn = jnp.where(odd == 1, np.float32(-1.0), np.float32(1.0))
    s = ps * sgn
    c4 = pc * sgn
    rp = jnp.concatenate([s[:, 0:4], c4[:, 0:4], s[:, 4:8], c4[:, 4:8],
                          s[:, 8:12], c4[:, 8:12], rel], axis=1)
    h = (
        jax.lax.dot_general(feat, w1f_ref[...], (((1,), (0,)), ((), ())),
                            preferred_element_type=jnp.float32)
        + jax.lax.dot_general(rp, w1rp_ref[...], (((1,), (0,)), ((), ())),
                              preferred_element_type=jnp.float32)
        + b1_ref[...]
    )
    h = jnp.maximum(h, 0.0)
    o = jax.lax.dot_general(h, w2_ref[...], (((1,), (0,)), ((), ())),
                            preferred_element_type=jnp.float32) + b2_ref[...]
    o3 = o.reshape(o.shape[0] // NSAMPLE, NSAMPLE, PATCH)
    o_ref[...] = jnp.max(o3, axis=1)


RBLK = 2048


def _tc_mlp(gathered, crep, W1f, W1rp, b1, W2, b2):
    nrow = B * S * NSAMPLE
    grid = (nrow // RBLK,)
    return pl.pallas_call(
        _mlp_body,
        grid=grid,
        in_specs=[
            pl.BlockSpec((RBLK, STEM), lambda i: (i, 0)),
            pl.BlockSpec((RBLK, 3), lambda i: (i, 0)),
            pl.BlockSpec((STEM, PATCH), lambda i: (0, 0)),
            pl.BlockSpec((PE_DIM + 3, PATCH), lambda i: (0, 0)),
            pl.BlockSpec((1, PATCH), lambda i: (0, 0)),
            pl.BlockSpec((PATCH, PATCH), lambda i: (0, 0)),
            pl.BlockSpec((1, PATCH), lambda i: (0, 0)),
        ],
        out_specs=pl.BlockSpec((RBLK // NSAMPLE, PATCH), lambda i: (i, 0)),
        out_shape=jax.ShapeDtypeStruct((B * S, PATCH), jnp.float32),
    )(gathered, crep, W1f, W1rp, b1, W2, b2)


def kernel(xyz, point_feature, patch_center, W1, b1, W2, b2):
    xyzT = jnp.transpose(xyz, (0, 2, 1)).reshape(B * 3, N)
    centT = jnp.transpose(patch_center, (0, 2, 1)).reshape(B * 3, S)
    neighbor_idx, flat_idx, rel = _sc_select(xyzT, centT)

    table = point_feature.reshape(B * N, STEM)
    rows = _sc_gather(table, flat_idx.reshape(-1))

    rel_flat = jnp.transpose(rel.reshape(B * S, 3, NSAMPLE),
                             (0, 2, 1)).reshape(B * S * NSAMPLE, 3)
    W1f = W1[:STEM]
    W1rp = jnp.concatenate([W1[STEM + 3:], W1[STEM:STEM + 3]], axis=0)
    pf = _tc_mlp(rows, rel_flat, W1f, W1rp, b1.reshape(1, PATCH), W2,
                 b2.reshape(1, PATCH))
    return pf.reshape(B, S, PATCH), neighbor_idx


# speedup vs baseline: 17.8221x; 1.1127x over previous
import dataclasses
import functools

import jax
import jax.numpy as jnp
import numpy as np
from jax import lax
from jax.experimental import pallas as pl
from jax.experimental.pallas import tpu as pltpu
from jax.experimental.pallas import tpu_sc as plsc

RADIUS = 0.2
NSAMPLE = 32
PE_DIM = 24
STEM = 128
PATCH = 256

B, N, S = 4, 8192, 512
GW = 128
NC, NS_SC, L = 2, 16, 16
NW = NC * NS_SC
CPW = (B * S) // NW
SBLK = S // (NW // B)
NVEC = N // L
GCHUNK = 256
ROWS_PW = (B * S * NSAMPLE) // NW


def _bf16r(v):
    u = plsc.bitcast(v, jnp.int32)
    r = (u + 0x7FFF + jnp.bitwise_and(jnp.right_shift(u, 16), 1)) & jnp.int32(-65536)
    return plsc.bitcast(r, jnp.float32)


def _sc_select(xyzT, centT):
    mesh = plsc.VectorSubcoreMesh(core_axis_name="c", subcore_axis_name="s")
    r2 = np.float32(RADIUS * RADIUS)
    cp = pltpu.CompilerParams()
    if "needs_layout_passes" in pltpu.CompilerParams.__dataclass_fields__:
        cp = dataclasses.replace(cp, needs_layout_passes=False)

    @functools.partial(
        pl.kernel,
        compiler_params=cp,
        out_type=(
            jax.ShapeDtypeStruct((B, S, NSAMPLE), jnp.int32),
            jax.ShapeDtypeStruct((B * S, NSAMPLE), jnp.int32),
            jax.ShapeDtypeStruct((B * S, 3 * NSAMPLE), jnp.float32),
        ),
        mesh=mesh,
        scratch_types=[
            pltpu.VMEM((1, N), jnp.float32),
            pltpu.VMEM((1, N), jnp.float32),
            pltpu.VMEM((1, N), jnp.float32),
            pltpu.VMEM((1, N), jnp.float32),
            pltpu.VMEM((1, N), jnp.float32),
            pltpu.VMEM((1, N), jnp.float32),
            pltpu.VMEM((1, N), jnp.float32),
            pltpu.VMEM((1, S), jnp.float32),
            pltpu.VMEM((1, S), jnp.float32),
            pltpu.VMEM((1, S), jnp.float32),
            pltpu.VMEM((128,), jnp.int32),
            pltpu.VMEM((SBLK, NSAMPLE), jnp.int32),
            pltpu.VMEM((SBLK, NSAMPLE), jnp.int32),
            pltpu.VMEM((SBLK, 3 * NSAMPLE), jnp.float32),
        ],
    )
    def sel_kernel(xyz_hbm, cent_hbm, nbr_hbm, flat_hbm, rel_hbm,
                   xv, yv, zv, nv, xrv, yrv, zrv, cxv, cyv, czv, cand, outv,
                   flatv, relv):
        wid = lax.axis_index("s") * NC + lax.axis_index("c")
        b = wid // (NW // B)
        s0 = (wid % (NW // B)) * SBLK
        pltpu.sync_copy(xyz_hbm.at[pl.ds(b * 3 + 0, 1)], xv)
        pltpu.sync_copy(xyz_hbm.at[pl.ds(b * 3 + 1, 1)], yv)
        pltpu.sync_copy(xyz_hbm.at[pl.ds(b * 3 + 2, 1)], zv)
        pltpu.sync_copy(cent_hbm.at[pl.ds(b * 3 + 0, 1)], cxv)
        pltpu.sync_copy(cent_hbm.at[pl.ds(b * 3 + 1, 1)], cyv)
        pltpu.sync_copy(cent_hbm.at[pl.ds(b * 3 + 2, 1)], czv)

        @pl.loop(0, N, step=L)
        def _(i):
            x = xv[0, pl.ds(i, L)]
            y = yv[0, pl.ds(i, L)]
            z = zv[0, pl.ds(i, L)]
            nv[0, pl.ds(i, L)] = (x * x + y * y) + z * z
            xrv[0, pl.ds(i, L)] = np.float32(-2.0) * _bf16r(x)
            yrv[0, pl.ds(i, L)] = np.float32(-2.0) * _bf16r(y)
            zrv[0, pl.ds(i, L)] = np.float32(-2.0) * _bf16r(z)

        iota16 = lax.iota(jnp.int32, L)

        @pl.loop(0, SBLK)
        def _center(j):
            jv = jnp.full((L,), j + s0, jnp.int32)
            zv16 = jnp.zeros((L,), jnp.int32)
            cx = plsc.load_gather(cxv, [zv16, jv])
            cy = plsc.load_gather(cyv, [zv16, jv])
            cz = plsc.load_gather(czv, [zv16, jv])
            cn = (cx * cx + cy * cy) + cz * cz
            cxr = _bf16r(cx)
            cyr = _bf16r(cy)
            czr = _bf16r(cz)

            def one_vec(nb, cur):
                x = xrv[0, pl.ds(nb, L)]
                y = yrv[0, pl.ds(nb, L)]
                z = zrv[0, pl.ds(nb, L)]
                xn = nv[0, pl.ds(nb, L)]
                dot2 = (cxr * x + cyr * y) + czr * z
                d2 = (dot2 + cn) + xn
                m = d2 <= r2
                mi = m.astype(jnp.int32)
                rank = jnp.cumsum(mi)
                plsc.store_scatter(cand, [rank + (cur - 1)], iota16 + nb,
                                   mask=m)
                return cur + jnp.sum(mi)

            def cond(c):
                i, cur = c
                return jnp.logical_and(cur < NSAMPLE, i < NVEC // 2)

            def body(c):
                i, cur = c
                nb = i * (2 * L)
                cur = one_vec(nb, cur)
                cur = one_vec(nb + L, cur)
                return i + 1, cur

            _, found = lax.while_loop(cond, body, (0, 0))
            foundv = jnp.full((L,), jnp.minimum(found, NSAMPLE), jnp.int32)
            first = plsc.load_gather(cand, [jnp.zeros((L,), jnp.int32)])
            firstv = jnp.where(foundv > 0, first, jnp.full((L,), N, jnp.int32))
            bo = jnp.full((L,), b * N, jnp.int32)
            for v in range(NSAMPLE // L):
                kio = lax.iota(jnp.int32, L) + v * L
                cur = cand[pl.ds(v * L, L)]
                sel = jnp.where(kio < foundv, cur, firstv)
                clp = jnp.minimum(sel, N - 1)
                outv[j, pl.ds(v * L, L)] = sel
                flatv[j, pl.ds(v * L, L)] = clp + bo
                gx = plsc.load_gather(xv, [zv16, clp])
                gy = plsc.load_gather(yv, [zv16, clp])
                gz = plsc.load_gather(zv, [zv16, clp])
                relv[j, pl.ds(0 * NSAMPLE + v * L, L)] = gx - cx
                relv[j, pl.ds(1 * NSAMPLE + v * L, L)] = gy - cy
                relv[j, pl.ds(2 * NSAMPLE + v * L, L)] = gz - cz

        pltpu.sync_copy(outv, nbr_hbm.at[b, pl.ds(s0, SBLK)])
        pltpu.sync_copy(flatv, flat_hbm.at[pl.ds(b * S + s0, SBLK)])
        pltpu.sync_copy(relv, rel_hbm.at[pl.ds(b * S + s0, SBLK)])

    return sel_kernel(xyzT, centT)


def _sc_gather(table, flat_idx):
    mesh = plsc.VectorSubcoreMesh(core_axis_name="c", subcore_axis_name="s")

    @functools.partial(
        pl.kernel,
        out_type=jax.ShapeDtypeStruct((B * S * NSAMPLE, GW), jnp.float32),
        mesh=mesh,
        scratch_types=[
            pltpu.VMEM((ROWS_PW,), jnp.int32),
            pltpu.VMEM((GCHUNK, GW), jnp.float32),
            pltpu.SemaphoreType.DMA,
        ],
    )
    def gather_kernel_seq(table_hbm, idx_hbm, out_hbm, idxv, rowsv, sem):
        wid = lax.axis_index("s") * NC + lax.axis_index("c")
        base = wid * ROWS_PW
        pltpu.sync_copy(idx_hbm.at[pl.ds(base, ROWS_PW)], idxv)

        @pl.loop(0, ROWS_PW // GCHUNK)
        def _(c):
            pltpu.async_copy(
                table_hbm.at[idxv.at[pl.ds(c * GCHUNK, GCHUNK)]],
                rowsv, sem).wait()
            pltpu.sync_copy(rowsv, out_hbm.at[pl.ds(base + c * GCHUNK, GCHUNK)])

    return gather_kernel_seq(table, flat_idx)


def _mlp_body(g_ref, r_ref, w1f_ref, w1rp_ref, b1_ref, w2_ref, b2_ref, o_ref):
    feat = g_ref[...]
    rel = r_ref[...]
    nf = PE_DIM // 6
    f4 = jnp.exp2(lax.broadcasted_iota(jnp.int32, (1, nf), 1).astype(jnp.float32))
    t = jnp.concatenate([rel[:, c:c + 1] * f4 for c in range(3)], axis=1)
    n = jnp.floor(t + np.float32(0.5))
    r = t - n
    r2 = r * r
    ps = ((((np.float32(0.07744687795639038) * r2
             + np.float32(-0.5981614589691162)) * r2
            + np.float32(2.550050973892212)) * r2
           + np.float32(-5.167707920074463)) * r2
          + np.float32(3.141592502593994)) * r
    pc = (((((np.float32(-0.024456139653921127) * r2
              + np.float32(0.2349717617034912)) * r2
             + np.float32(-1.335218906402588)) * r2
            + np.float32(4.058709621429443)) * r2
           + np.float32(-4.934802055358887)) * r2
          + np.float32(1.0))
    odd = jnp.bitwise_and(n.astype(jnp.int32), 1)
    sgn = jnp.where(odd == 1, np.float32(-1.0), np.float32(1.0))
    s = ps * sgn
    c4 = pc * sgn
    rp = jnp.concatenate([s[:, 0:4], c4[:, 0:4], s[:, 4:8], c4[:, 4:8],
                          s[:, 8:12], c4[:, 8:12], rel], axis=1)
    h = (
        jax.lax.dot_general(feat, w1f_ref[...], (((1,), (0,)), ((), ())),
                            preferred_element_type=jnp.float32)
        + jax.lax.dot_general(rp, w1rp_ref[...], (((1,), (0,)), ((), ())),
                              preferred_element_type=jnp.float32)
        + b1_ref[...]
    )
    h = jnp.maximum(h, 0.0)
    o = jax.lax.dot_general(h, w2_ref[...], (((1,), (0,)), ((), ())),
                            preferred_element_type=jnp.float32) + b2_ref[...]
    o3 = o.reshape(o.shape[0] // NSAMPLE, NSAMPLE, PATCH)
    o_ref[...] = jnp.max(o3, axis=1)


RBLK = 2048


def _tc_mlp(gathered, crep, W1f, W1rp, b1, W2, b2):
    nrow = B * S * NSAMPLE
    grid = (nrow // RBLK,)
    return pl.pallas_call(
        _mlp_body,
        grid=grid,
        in_specs=[
            pl.BlockSpec((RBLK, STEM), lambda i: (i, 0)),
            pl.BlockSpec((RBLK, 3), lambda i: (i, 0)),
            pl.BlockSpec((STEM, PATCH), lambda i: (0, 0)),
            pl.BlockSpec((PE_DIM + 3, PATCH), lambda i: (0, 0)),
            pl.BlockSpec((1, PATCH), lambda i: (0, 0)),
            pl.BlockSpec((PATCH, PATCH), lambda i: (0, 0)),
            pl.BlockSpec((1, PATCH), lambda i: (0, 0)),
        ],
        out_specs=pl.BlockSpec((RBLK // NSAMPLE, PATCH), lambda i: (i, 0)),
        out_shape=jax.ShapeDtypeStruct((B * S, PATCH), jnp.float32),
    )(gathered, crep, W1f, W1rp, b1, W2, b2)


def kernel(xyz, point_feature, patch_center, W1, b1, W2, b2):
    xyzT = jnp.transpose(xyz, (0, 2, 1)).reshape(B * 3, N)
    centT = jnp.transpose(patch_center, (0, 2, 1)).reshape(B * 3, S)
    neighbor_idx, flat_idx, rel = _sc_select(xyzT, centT)

    table = point_feature.reshape(B * N, STEM)
    rows = _sc_gather(table, flat_idx.reshape(-1))

    rel_flat = jnp.transpose(rel.reshape(B * S, 3, NSAMPLE),
                             (0, 2, 1)).reshape(B * S * NSAMPLE, 3)
    W1f = W1[:STEM]
    W1rp = jnp.concatenate([W1[STEM + 3:], W1[STEM:STEM + 3]], axis=0)
    pf = _tc_mlp(rows, rel_flat, W1f, W1rp, b1.reshape(1, PATCH), W2,
                 b2.reshape(1, PATCH))
    return pf.reshape(B, S, PATCH), neighbor_idx


# speedup vs baseline: 18.0085x; 1.0105x over previous
import dataclasses
import functools

import jax
import jax.numpy as jnp
import numpy as np
from jax import lax
from jax.experimental import pallas as pl
from jax.experimental.pallas import tpu as pltpu
from jax.experimental.pallas import tpu_sc as plsc

RADIUS = 0.2
NSAMPLE = 32
PE_DIM = 24
STEM = 128
PATCH = 256

B, N, S = 4, 8192, 512
GW = 128
NC, NS_SC, L = 2, 16, 16
NW = NC * NS_SC
CPW = (B * S) // NW
SBLK = S // (NW // B)
NVEC = N // L
GCHUNK = 256
ROWS_PW = (B * S * NSAMPLE) // NW


def _bf16r(v):
    u = plsc.bitcast(v, jnp.int32)
    r = (u + 0x7FFF + jnp.bitwise_and(jnp.right_shift(u, 16), 1)) & jnp.int32(-65536)
    return plsc.bitcast(r, jnp.float32)


def _sc_select(xyzT, centT):
    mesh = plsc.VectorSubcoreMesh(core_axis_name="c", subcore_axis_name="s")
    r2 = np.float32(RADIUS * RADIUS)
    cp = pltpu.CompilerParams()
    if "needs_layout_passes" in pltpu.CompilerParams.__dataclass_fields__:
        cp = dataclasses.replace(cp, needs_layout_passes=False)

    @functools.partial(
        pl.kernel,
        compiler_params=cp,
        out_type=(
            jax.ShapeDtypeStruct((B, S, NSAMPLE), jnp.int32),
            jax.ShapeDtypeStruct((B * S, NSAMPLE), jnp.int32),
            jax.ShapeDtypeStruct((B * S, 3 * NSAMPLE), jnp.float32),
        ),
        mesh=mesh,
        scratch_types=[
            pltpu.VMEM((1, N), jnp.float32),
            pltpu.VMEM((1, N), jnp.float32),
            pltpu.VMEM((1, N), jnp.float32),
            pltpu.VMEM((1, N), jnp.float32),
            pltpu.VMEM((1, N), jnp.float32),
            pltpu.VMEM((1, N), jnp.float32),
            pltpu.VMEM((1, N), jnp.float32),
            pltpu.VMEM((1, S), jnp.float32),
            pltpu.VMEM((1, S), jnp.float32),
            pltpu.VMEM((1, S), jnp.float32),
            pltpu.VMEM((128,), jnp.int32),
            pltpu.VMEM((SBLK, NSAMPLE), jnp.int32),
            pltpu.VMEM((SBLK, NSAMPLE), jnp.int32),
            pltpu.VMEM((SBLK, 3 * NSAMPLE), jnp.float32),
        ],
    )
    def sel_kernel(xyz_hbm, cent_hbm, nbr_hbm, flat_hbm, rel_hbm,
                   xv, yv, zv, nv, xrv, yrv, zrv, cxv, cyv, czv, cand, outv,
                   flatv, relv):
        wid = lax.axis_index("s") * NC + lax.axis_index("c")
        b = wid // (NW // B)
        s0 = (wid % (NW // B)) * SBLK
        pltpu.sync_copy(xyz_hbm.at[pl.ds(b * 3 + 0, 1)], xv)
        pltpu.sync_copy(xyz_hbm.at[pl.ds(b * 3 + 1, 1)], yv)
        pltpu.sync_copy(xyz_hbm.at[pl.ds(b * 3 + 2, 1)], zv)
        pltpu.sync_copy(cent_hbm.at[pl.ds(b * 3 + 0, 1)], cxv)
        pltpu.sync_copy(cent_hbm.at[pl.ds(b * 3 + 1, 1)], cyv)
        pltpu.sync_copy(cent_hbm.at[pl.ds(b * 3 + 2, 1)], czv)

        @pl.loop(0, N, step=L)
        def _(i):
            x = xv[0, pl.ds(i, L)]
            y = yv[0, pl.ds(i, L)]
            z = zv[0, pl.ds(i, L)]
            nv[0, pl.ds(i, L)] = (x * x + y * y) + z * z
            xrv[0, pl.ds(i, L)] = np.float32(-2.0) * _bf16r(x)
            yrv[0, pl.ds(i, L)] = np.float32(-2.0) * _bf16r(y)
            zrv[0, pl.ds(i, L)] = np.float32(-2.0) * _bf16r(z)

        iota16 = lax.iota(jnp.int32, L)

        @pl.loop(0, SBLK)
        def _center(j):
            jv = jnp.full((L,), j + s0, jnp.int32)
            zv16 = jnp.zeros((L,), jnp.int32)
            cx = plsc.load_gather(cxv, [zv16, jv])
            cy = plsc.load_gather(cyv, [zv16, jv])
            cz = plsc.load_gather(czv, [zv16, jv])
            cn = (cx * cx + cy * cy) + cz * cz
            cxr = _bf16r(cx)
            cyr = _bf16r(cy)
            czr = _bf16r(cz)

            def one_vec(nb, cur):
                x = xrv[0, pl.ds(nb, L)]
                y = yrv[0, pl.ds(nb, L)]
                z = zrv[0, pl.ds(nb, L)]
                xn = nv[0, pl.ds(nb, L)]
                dot2 = (cxr * x + cyr * y) + czr * z
                d2 = (dot2 + cn) + xn
                m = d2 <= r2
                mi = m.astype(jnp.int32)
                rank = jnp.cumsum(mi)
                plsc.store_scatter(cand, [rank + (cur - 1)], iota16 + nb,
                                   mask=m)
                return cur + jnp.sum(mi)

            def cond(c):
                i, cur = c
                return jnp.logical_and(cur < NSAMPLE, i < NVEC // 2)

            def body(c):
                i, cur = c
                nb = i * (2 * L)
                cur = one_vec(nb, cur)
                cur = one_vec(nb + L, cur)
                return i + 1, cur

            _, found = lax.while_loop(cond, body, (0, 0))
            foundv = jnp.full((L,), jnp.minimum(found, NSAMPLE), jnp.int32)
            first = plsc.load_gather(cand, [jnp.zeros((L,), jnp.int32)])
            firstv = jnp.where(foundv > 0, first, jnp.full((L,), N, jnp.int32))
            bo = jnp.full((L,), b * N, jnp.int32)
            for v in range(NSAMPLE // L):
                kio = lax.iota(jnp.int32, L) + v * L
                cur = cand[pl.ds(v * L, L)]
                sel = jnp.where(kio < foundv, cur, firstv)
                clp = jnp.minimum(sel, N - 1)
                outv[j, pl.ds(v * L, L)] = sel
                flatv[j, pl.ds(v * L, L)] = clp + bo
                gx = plsc.load_gather(xv, [zv16, clp])
                gy = plsc.load_gather(yv, [zv16, clp])
                gz = plsc.load_gather(zv, [zv16, clp])
                relv[j, pl.ds(0 * NSAMPLE + v * L, L)] = gx - cx
                relv[j, pl.ds(1 * NSAMPLE + v * L, L)] = gy - cy
                relv[j, pl.ds(2 * NSAMPLE + v * L, L)] = gz - cz

        pltpu.sync_copy(outv, nbr_hbm.at[b, pl.ds(s0, SBLK)])
        pltpu.sync_copy(flatv, flat_hbm.at[pl.ds(b * S + s0, SBLK)])
        pltpu.sync_copy(relv, rel_hbm.at[pl.ds(b * S + s0, SBLK)])

    return sel_kernel(xyzT, centT)


def _sc_gather(table, flat_idx):
    mesh = plsc.VectorSubcoreMesh(core_axis_name="c", subcore_axis_name="s")

    nchunk = ROWS_PW // GCHUNK

    @functools.partial(
        pl.kernel,
        out_type=jax.ShapeDtypeStruct((B * S * NSAMPLE, GW), jnp.float32),
        mesh=mesh,
        scratch_types=[
            pltpu.VMEM((ROWS_PW,), jnp.int32),
            pltpu.VMEM((GCHUNK, GW), jnp.float32),
            pltpu.VMEM((GCHUNK, GW), jnp.float32),
            pltpu.SemaphoreType.DMA,
            pltpu.SemaphoreType.DMA,
            pltpu.SemaphoreType.DMA,
            pltpu.SemaphoreType.DMA,
        ],
    )
    def gather_kernel_db(table_hbm, idx_hbm, out_hbm, idxv, buf0, buf1,
                         g0, g1, o0, o1):
        wid = lax.axis_index("s") * NC + lax.axis_index("c")
        base = wid * ROWS_PW
        pltpu.sync_copy(idx_hbm.at[pl.ds(base, ROWS_PW)], idxv)
        bufs = (buf0, buf1)
        gsems = (g0, g1)
        osems = (o0, o1)

        def gstart(c):
            return pltpu.async_copy(
                table_hbm.at[idxv.at[pl.ds(c * GCHUNK, GCHUNK)]],
                bufs[c % 2], gsems[c % 2])

        def ostart(c):
            return pltpu.async_copy(
                bufs[c % 2], out_hbm.at[pl.ds(base + c * GCHUNK, GCHUNK)],
                osems[c % 2])

        gh = {0: gstart(0), 1: gstart(1)}
        oh = {}
        for c in range(nchunk):
            gh[c].wait()
            oh[c] = ostart(c)
            if c + 2 < nchunk:
                oh[c].wait()
                gh[c + 2] = gstart(c + 2)
        oh[nchunk - 2].wait()
        oh[nchunk - 1].wait()

    return gather_kernel_db(table, flat_idx)


def _mlp_body(g_ref, r_ref, w1f_ref, w1rp_ref, b1_ref, w2_ref, b2_ref, o_ref):
    feat = g_ref[...]
    rel = r_ref[...]
    nf = PE_DIM // 6
    f4 = jnp.exp2(lax.broadcasted_iota(jnp.int32, (1, nf), 1).astype(jnp.float32))
    t = jnp.concatenate([rel[:, c:c + 1] * f4 for c in range(3)], axis=1)
    n = jnp.floor(t + np.float32(0.5))
    r = t - n
    r2 = r * r
    ps = ((((np.float32(0.07744687795639038) * r2
             + np.float32(-0.5981614589691162)) * r2
            + np.float32(2.550050973892212)) * r2
           + np.float32(-5.167707920074463)) * r2
          + np.float32(3.141592502593994)) * r
    pc = (((((np.float32(-0.024456139653921127) * r2
              + np.float32(0.2349717617034912)) * r2
             + np.float32(-1.335218906402588)) * r2
            + np.float32(4.058709621429443)) * r2
           + np.float32(-4.934802055358887)) * r2
          + np.float32(1.0))
    odd = jnp.bitwise_and(n.astype(jnp.int32), 1)
    sgn = jnp.where(odd == 1, np.float32(-1.0), np.float32(1.0))
    s = ps * sgn
    c4 = pc * sgn
    rp = jnp.concatenate([s[:, 0:4], c4[:, 0:4], s[:, 4:8], c4[:, 4:8],
                          s[:, 8:12], c4[:, 8:12], rel], axis=1)
    h = (
        jax.lax.dot_general(feat, w1f_ref[...], (((1,), (0,)), ((), ())),
                            preferred_element_type=jnp.float32)
        + jax.lax.dot_general(rp, w1rp_ref[...], (((1,), (0,)), ((), ())),
                              preferred_element_type=jnp.float32)
        + b1_ref[...]
    )
    h = jnp.maximum(h, 0.0)
    o = jax.lax.dot_general(h, w2_ref[...], (((1,), (0,)), ((), ())),
                            preferred_element_type=jnp.float32) + b2_ref[...]
    o3 = o.reshape(o.shape[0] // NSAMPLE, NSAMPLE, PATCH)
    o_ref[...] = jnp.max(o3, axis=1)


RBLK = 4096


def _tc_mlp(gathered, crep, W1f, W1rp, b1, W2, b2):
    nrow = B * S * NSAMPLE
    grid = (nrow // RBLK,)
    return pl.pallas_call(
        _mlp_body,
        grid=grid,
        in_specs=[
            pl.BlockSpec((RBLK, STEM), lambda i: (i, 0)),
            pl.BlockSpec((RBLK, 3), lambda i: (i, 0)),
            pl.BlockSpec((STEM, PATCH), lambda i: (0, 0)),
            pl.BlockSpec((PE_DIM + 3, PATCH), lambda i: (0, 0)),
            pl.BlockSpec((1, PATCH), lambda i: (0, 0)),
            pl.BlockSpec((PATCH, PATCH), lambda i: (0, 0)),
            pl.BlockSpec((1, PATCH), lambda i: (0, 0)),
        ],
        out_specs=pl.BlockSpec((RBLK // NSAMPLE, PATCH), lambda i: (i, 0)),
        out_shape=jax.ShapeDtypeStruct((B * S, PATCH), jnp.float32),
    )(gathered, crep, W1f, W1rp, b1, W2, b2)


def kernel(xyz, point_feature, patch_center, W1, b1, W2, b2):
    xyzT = jnp.transpose(xyz, (0, 2, 1)).reshape(B * 3, N)
    centT = jnp.transpose(patch_center, (0, 2, 1)).reshape(B * 3, S)
    neighbor_idx, flat_idx, rel = _sc_select(xyzT, centT)

    table = point_feature.reshape(B * N, STEM)
    rows = _sc_gather(table, flat_idx.reshape(-1))

    rel_flat = jnp.transpose(rel.reshape(B * S, 3, NSAMPLE),
                             (0, 2, 1)).reshape(B * S * NSAMPLE, 3)
    W1f = W1[:STEM]
    W1rp = jnp.concatenate([W1[STEM + 3:], W1[STEM:STEM + 3]], axis=0)
    pf = _tc_mlp(rows, rel_flat, W1f, W1rp, b1.reshape(1, PATCH), W2,
                 b2.reshape(1, PATCH))
    return pf.reshape(B, S, PATCH), neighbor_idx
